# baseline (speedup 1.0000x reference)
import functools

import jax
import jax.numpy as jnp
from jax import lax
from jax.experimental import pallas as pl
from jax.experimental.pallas import tpu as pltpu
from jax.experimental.pallas import tpu_sc as plsc

N = 10000
E = 320000
H = 128
OUT = 128
L = 3
G = 64
VPAD = 104

NC = 2
NS = 16
NW = NC * NS
FPT = H // NW
CH = 1024
NCH = E // CH
NG = CH // 16

_F32 = jnp.float32


def _feat_major(x, w, b_col):
    r = lax.dot_general(w, x, (((0,), (1,)), ((), ())),
                        preferred_element_type=jnp.float32)
    return r if b_col is None else r + b_col


def _tc_init_body(z_ref, emb_ref, w1a_ref, w1b_ref, b1_ref,
                  h_ref, at_ref, bt_ref):
    iota = lax.broadcasted_iota(jnp.int32, (N, VPAD), 1)
    oh = (z_ref[...] == iota).astype(jnp.float32)
    h = jnp.dot(oh, emb_ref[...], preferred_element_type=jnp.float32)
    h_ref[...] = h
    at_ref[...] = _feat_major(h, w1a_ref[...], b1_ref[...])
    bt_ref[...] = _feat_major(h, w1b_ref[...], None)


def _tc_update_core(h, st_ref, deg_ref, w2_ref, b2_ref, u1a_ref, u1b_ref,
                    ub1_ref, u2_ref, ub2_ref):
    agg = lax.dot_general(st_ref[...], w2_ref[...], (((0,), (0,)), ((), ())),
                          preferred_element_type=jnp.float32)
    deg_col = lax.dot_general(deg_ref[...], jnp.ones((NW, 1), jnp.float32),
                              (((0,), (0,)), ((), ())),
                              preferred_element_type=jnp.float32)
    agg = agg + deg_col * b2_ref[...]
    t = jnp.maximum(
        jnp.dot(h, u1a_ref[...], preferred_element_type=jnp.float32)
        + jnp.dot(agg, u1b_ref[...], preferred_element_type=jnp.float32)
        + ub1_ref[...], 0.0)
    return h + jnp.dot(t, u2_ref[...],
                       preferred_element_type=jnp.float32) + ub2_ref[...]


def _tc_update_body(h_ref, st_ref, deg_ref, w2_ref, b2_ref, u1a_ref, u1b_ref,
                    ub1_ref, u2_ref, ub2_ref, w1a_ref, w1b_ref, b1_ref,
                    hn_ref, at_ref, bt_ref):
    hn = _tc_update_core(h_ref[...], st_ref, deg_ref, w2_ref, b2_ref,
                         u1a_ref, u1b_ref, ub1_ref, u2_ref, ub2_ref)
    hn_ref[...] = hn
    at_ref[...] = _feat_major(hn, w1a_ref[...], b1_ref[...])
    bt_ref[...] = _feat_major(hn, w1b_ref[...], None)


def _tc_final_body(h_ref, st_ref, deg_ref, w2_ref, b2_ref, u1a_ref, u1b_ref,
                   ub1_ref, u2_ref, ub2_ref, batch_ref, hw1_ref, hb1_ref,
                   hw2_ref, hb2_ref, out_ref):
    hn = _tc_update_core(h_ref[...], st_ref, deg_ref, w2_ref, b2_ref,
                         u1a_ref, u1b_ref, ub1_ref, u2_ref, ub2_ref)
    iota = lax.broadcasted_iota(jnp.int32, (N, G), 1)
    oh = (batch_ref[...] == iota).astype(jnp.float32)
    dimnum = (((0,), (0,)), ((), ()))
    pooled = lax.dot_general(oh, hn, dimnum,
                             preferred_element_type=jnp.float32)
    cnt = lax.dot_general(oh, jnp.ones((N, 1), jnp.float32), dimnum,
                          preferred_element_type=jnp.float32)
    pm = pooled / jnp.maximum(cnt, 1.0)
    t = jnp.maximum(
        jnp.dot(pm, hw1_ref[...], preferred_element_type=jnp.float32)
        + hb1_ref[...], 0.0)
    out_ref[...] = jnp.dot(t, hw2_ref[...],
                           preferred_element_type=jnp.float32) + hb2_ref[...]


_nd = jax.ShapeDtypeStruct((N, H), _F32)
_fm = jax.ShapeDtypeStruct((H, N), _F32)

_tc_init = pl.pallas_call(_tc_init_body, out_shape=[_nd, _fm, _fm])
_tc_update = pl.pallas_call(_tc_update_body, out_shape=[_nd, _fm, _fm])
_tc_final = pl.pallas_call(_tc_final_body,
                           out_shape=jax.ShapeDtypeStruct((G, OUT), _F32))


def _sc_edge_body(a3_hbm, b3_hbm, src_hbm, dst_hbm, ea_hbm, w1e_hbm,
                  st_out,
                  idx_sv, idx_dv, ea_v, a_rows, b_rows, acc, w1e_v,
                  sem0, sem1):
    c = lax.axis_index("c")
    s = lax.axis_index("s")
    wid = s * NC + c
    zero16 = jnp.zeros((16,), _F32)
    one16 = jnp.ones((16,), _F32)
    sems = [sem0, sem1]

    def zrow(i, _):
        for f in range(FPT):
            acc[f, pl.ds(i * 16, 16)] = zero16
        return 0
    lax.fori_loop(0, N // 16, zrow, 0)

    pltpu.sync_copy(a3_hbm.at[wid], a_rows)
    pltpu.sync_copy(b3_hbm.at[wid], b_rows)
    pltpu.sync_copy(w1e_hbm, w1e_v)
    w1ef = [plsc.load_gather(w1e_v, [jnp.full((16,), 0, jnp.int32)
                                     + (wid * FPT + f)])
            for f in range(FPT)]
    rowf = [jnp.full((16,), f, jnp.int32) for f in range(FPT)]

    def issue(slot, g):
        base = g * CH
        pltpu.make_async_copy(src_hbm.at[pl.ds(base, CH)],
                              idx_sv.at[slot], sems[0]).start()
        pltpu.make_async_copy(dst_hbm.at[pl.ds(base, CH)],
                              idx_dv.at[slot], sems[0]).start()
        pltpu.make_async_copy(ea_hbm.at[pl.ds(base, CH)],
                              ea_v.at[slot], sems[1]).start()

    def drain(slot, g):
        base = g * CH
        pltpu.make_async_copy(src_hbm.at[pl.ds(base, CH)],
                              idx_sv.at[slot], sems[0]).wait()
        pltpu.make_async_copy(dst_hbm.at[pl.ds(base, CH)],
                              idx_dv.at[slot], sems[0]).wait()
        pltpu.make_async_copy(ea_hbm.at[pl.ds(base, CH)],
                              ea_v.at[slot], sems[1]).wait()

    issue(0, 0)

    def chunk(g, _):
        slot = lax.rem(g, 2)

        @pl.when(g + 1 < NCH)
        def _pref():
            issue(1 - slot, g + 1)
        drain(slot, g)

        @plsc.parallel_loop(0, NG, unroll=16)
        def group(i):
            sl = pl.ds(i * 16, 16)
            src16 = idx_sv[slot, sl]
            dst16 = idx_dv[slot, sl]
            ea16 = ea_v[slot, sl]
            for f in range(FPT):
                a16 = plsc.load_gather(a_rows, [rowf[f], src16])
                b16 = plsc.load_gather(b_rows, [rowf[f], dst16])
                r = jnp.maximum(a16 + b16 + ea16 * w1ef[f], 0.0)
                plsc.addupdate_scatter(acc, [rowf[f], dst16], r)
        return 0
    lax.fori_loop(0, NCH, chunk, 0)

    pltpu.sync_copy(acc, st_out.at[wid])


@functools.cache
def _sc_edge():
  return pl.kernel(
    _sc_edge_body,
    out_type=jax.ShapeDtypeStruct((NW, FPT, N), _F32),
    mesh=plsc.VectorSubcoreMesh(core_axis_name="c", subcore_axis_name="s",
                                num_cores=NC, num_subcores=NS),
    compiler_params=pltpu.CompilerParams(needs_layout_passes=False,
                                         internal_scratch_in_bytes=16 * 1024),
    scratch_types=[
        pltpu.VMEM((2, CH), jnp.int32),
        pltpu.VMEM((2, CH), jnp.int32),
        pltpu.VMEM((2, CH), _F32),
        pltpu.VMEM((FPT, N), _F32),
        pltpu.VMEM((FPT, N), _F32),
        pltpu.VMEM((FPT, N), _F32),
        pltpu.VMEM((H,), _F32),
        pltpu.SemaphoreType.DMA,
        pltpu.SemaphoreType.DMA,
    ],
  )


CHD = 2000
EPW = E // NW


def _sc_deg_body(dst_hbm, deg_out, idxd_v, dacc):
    c = lax.axis_index("c")
    s = lax.axis_index("s")
    wid = s * NC + c
    zero16 = jnp.zeros((16,), _F32)
    one16 = jnp.ones((16,), _F32)
    row0 = jnp.full((16,), 0, jnp.int32)

    def zrow(i, _):
        dacc[0, pl.ds(i * 16, 16)] = zero16
        return 0
    lax.fori_loop(0, N // 16, zrow, 0)

    def chunk(q, _):
        pltpu.sync_copy(dst_hbm.at[pl.ds(wid * EPW + q * CHD, CHD)], idxd_v)

        def group(i, _):
            dst16 = idxd_v[pl.ds(i * 16, 16)]
            plsc.addupdate_scatter(dacc, [row0, dst16], one16)
            return 0
        lax.fori_loop(0, CHD // 16, group, 0)
        return 0
    lax.fori_loop(0, EPW // CHD, chunk, 0)
    pltpu.sync_copy(dacc, deg_out.at[wid])


@functools.cache
def _sc_deg():
  return pl.kernel(
    _sc_deg_body,
    out_type=jax.ShapeDtypeStruct((NW, 1, N), _F32),
    mesh=plsc.VectorSubcoreMesh(core_axis_name="c", subcore_axis_name="s",
                                num_cores=NC, num_subcores=NS),
    compiler_params=pltpu.CompilerParams(needs_layout_passes=False,
                                         internal_scratch_in_bytes=32 * 1024),
    scratch_types=[
        pltpu.VMEM((CHD,), jnp.int32),
        pltpu.VMEM((1, N), _F32),
    ],
  )


def kernel(z, edge_index, edge_attr, batch, embed,
           msg_w1, msg_b1, msg_w2, msg_b2,
           upd_w1, upd_b1, upd_w2, upd_b2,
           head_w1, head_b1, head_w2, head_b2):
    z2 = z.astype(jnp.int32).reshape(N, 1)
    src = edge_index[0].astype(jnp.int32)
    dst = edge_index[1].astype(jnp.int32)
    ea = edge_attr.reshape(E).astype(jnp.float32)
    batch2 = batch.astype(jnp.int32).reshape(N, 1)
    emb_p = jnp.pad(embed, ((0, VPAD - embed.shape[0]), (0, 0)))

    w1a = [msg_w1[l, :H] for l in range(L)]
    w1b = [msg_w1[l, H:2 * H] for l in range(L)]
    w1e = [msg_w1[l, 2 * H] for l in range(L)]
    b1c = [msg_b1[l].reshape(H, 1) for l in range(L)]
    w2 = [msg_w2[l] for l in range(L)]
    b2 = [msg_b2[l].reshape(1, H) for l in range(L)]
    u1a = [upd_w1[l, :H] for l in range(L)]
    u1b = [upd_w1[l, H:] for l in range(L)]
    ub1 = [upd_b1[l].reshape(1, H) for l in range(L)]
    u2 = [upd_w2[l] for l in range(L)]
    ub2 = [upd_b2[l].reshape(1, H) for l in range(L)]

    h, at, bt = _tc_init(z2, emb_p, w1a[0], w1b[0], b1c[0])
    deg32 = _sc_deg()(dst).reshape(NW, N)
    out = None
    for l in range(L):
        st3 = _sc_edge()(at.reshape(NW, FPT, N), bt.reshape(NW, FPT, N),
                         src, dst, ea, w1e[l])
        st = st3.reshape(H, N)
        if l + 1 < L:
            h, at, bt = _tc_update(h, st, deg32, w2[l], b2[l], u1a[l], u1b[l],
                                   ub1[l], u2[l], ub2[l],
                                   w1a[l + 1], w1b[l + 1], b1c[l + 1])
        else:
            out = _tc_final(h, st, deg32, w2[l], b2[l], u1a[l], u1b[l],
                            ub1[l], u2[l], ub2[l], batch2,
                            head_w1, head_b1.reshape(1, H),
                            head_w2, head_b2.reshape(1, OUT))
    return out


# speedup vs baseline: 1.1638x; 1.1638x over previous
import functools

import jax
import jax.numpy as jnp
from jax import lax
from jax.experimental import pallas as pl
from jax.experimental.pallas import tpu as pltpu
from jax.experimental.pallas import tpu_sc as plsc

N = 10000
E = 320000
H = 128
OUT = 128
L = 3
G = 64
VPAD = 104

NC = 2
NS = 16
NW = NC * NS
FPT = H // NW
CH = 1024
NCH = E // CH
NG = CH // 16

_F32 = jnp.float32


def _feat_major(x, w, b_col):
    r = lax.dot_general(w, x, (((0,), (1,)), ((), ())),
                        preferred_element_type=jnp.float32)
    return r if b_col is None else r + b_col


def _tc_init_body(z_ref, emb_ref, w1a_ref, w1b_ref, b1_ref,
                  h_ref, at_ref, bt_ref):
    iota = lax.broadcasted_iota(jnp.int32, (N, VPAD), 1)
    oh = (z_ref[...] == iota).astype(jnp.float32)
    h = jnp.dot(oh, emb_ref[...], preferred_element_type=jnp.float32)
    h_ref[...] = h
    at_ref[...] = _feat_major(h, w1a_ref[...], b1_ref[...])
    bt_ref[...] = _feat_major(h, w1b_ref[...], None)


def _tc_update_core(h, st_ref, deg_ref, w2_ref, b2_ref, u1a_ref, u1b_ref,
                    ub1_ref, u2_ref, ub2_ref):
    agg = lax.dot_general(st_ref[...], w2_ref[...], (((0,), (0,)), ((), ())),
                          preferred_element_type=jnp.float32)
    deg_col = lax.dot_general(deg_ref[...], jnp.ones((NW, 1), jnp.float32),
                              (((0,), (0,)), ((), ())),
                              preferred_element_type=jnp.float32)
    agg = agg + deg_col * b2_ref[...]
    t = jnp.maximum(
        jnp.dot(h, u1a_ref[...], preferred_element_type=jnp.float32)
        + jnp.dot(agg, u1b_ref[...], preferred_element_type=jnp.float32)
        + ub1_ref[...], 0.0)
    return h + jnp.dot(t, u2_ref[...],
                       preferred_element_type=jnp.float32) + ub2_ref[...]


def _tc_update_body(h_ref, st_ref, deg_ref, w2_ref, b2_ref, u1a_ref, u1b_ref,
                    ub1_ref, u2_ref, ub2_ref, w1a_ref, w1b_ref, b1_ref,
                    hn_ref, at_ref, bt_ref):
    hn = _tc_update_core(h_ref[...], st_ref, deg_ref, w2_ref, b2_ref,
                         u1a_ref, u1b_ref, ub1_ref, u2_ref, ub2_ref)
    hn_ref[...] = hn
    at_ref[...] = _feat_major(hn, w1a_ref[...], b1_ref[...])
    bt_ref[...] = _feat_major(hn, w1b_ref[...], None)


def _tc_final_body(h_ref, st_ref, deg_ref, w2_ref, b2_ref, u1a_ref, u1b_ref,
                   ub1_ref, u2_ref, ub2_ref, batch_ref, hw1_ref, hb1_ref,
                   hw2_ref, hb2_ref, out_ref):
    hn = _tc_update_core(h_ref[...], st_ref, deg_ref, w2_ref, b2_ref,
                         u1a_ref, u1b_ref, ub1_ref, u2_ref, ub2_ref)
    iota = lax.broadcasted_iota(jnp.int32, (N, G), 1)
    oh = (batch_ref[...] == iota).astype(jnp.float32)
    dimnum = (((0,), (0,)), ((), ()))
    pooled = lax.dot_general(oh, hn, dimnum,
                             preferred_element_type=jnp.float32)
    cnt = lax.dot_general(oh, jnp.ones((N, 1), jnp.float32), dimnum,
                          preferred_element_type=jnp.float32)
    pm = pooled / jnp.maximum(cnt, 1.0)
    t = jnp.maximum(
        jnp.dot(pm, hw1_ref[...], preferred_element_type=jnp.float32)
        + hb1_ref[...], 0.0)
    out_ref[...] = jnp.dot(t, hw2_ref[...],
                           preferred_element_type=jnp.float32) + hb2_ref[...]


_nd = jax.ShapeDtypeStruct((N, H), _F32)
_fm = jax.ShapeDtypeStruct((H, N), _F32)

_tc_init = pl.pallas_call(_tc_init_body, out_shape=[_nd, _fm, _fm])
_tc_update = pl.pallas_call(_tc_update_body, out_shape=[_nd, _fm, _fm])
_tc_final = pl.pallas_call(_tc_final_body,
                           out_shape=jax.ShapeDtypeStruct((G, OUT), _F32))


def _sc_edge_body(a3_hbm, b3_hbm, src_hbm, dst_hbm, ea_hbm, w1e_hbm,
                  st_out,
                  idx_sv, idx_dv, ea_v, a_rows, b_rows, acc, w1e_v,
                  sem0, sem1):
    c = lax.axis_index("c")
    s = lax.axis_index("s")
    wid = s * NC + c
    zero16 = jnp.zeros((16,), _F32)
    one16 = jnp.ones((16,), _F32)
    sems = [sem0, sem1]

    def zrow(i, _):
        for f in range(FPT):
            acc[f, pl.ds(i * 16, 16)] = zero16
        return 0
    lax.fori_loop(0, N // 16, zrow, 0)

    pltpu.sync_copy(a3_hbm.at[wid], a_rows)
    pltpu.sync_copy(b3_hbm.at[wid], b_rows)
    pltpu.sync_copy(w1e_hbm, w1e_v)
    w1ef = [plsc.load_gather(w1e_v, [jnp.full((16,), 0, jnp.int32)
                                     + (wid * FPT + f)])
            for f in range(FPT)]
    rowf = [jnp.full((16,), f, jnp.int32) for f in range(FPT)]

    def issue(slot, g):
        base = g * CH
        pltpu.make_async_copy(src_hbm.at[pl.ds(base, CH)],
                              idx_sv.at[slot], sems[0]).start()
        pltpu.make_async_copy(dst_hbm.at[pl.ds(base, CH)],
                              idx_dv.at[slot], sems[0]).start()
        pltpu.make_async_copy(ea_hbm.at[pl.ds(base, CH)],
                              ea_v.at[slot], sems[1]).start()

    def drain(slot, g):
        base = g * CH
        pltpu.make_async_copy(src_hbm.at[pl.ds(base, CH)],
                              idx_sv.at[slot], sems[0]).wait()
        pltpu.make_async_copy(dst_hbm.at[pl.ds(base, CH)],
                              idx_dv.at[slot], sems[0]).wait()
        pltpu.make_async_copy(ea_hbm.at[pl.ds(base, CH)],
                              ea_v.at[slot], sems[1]).wait()

    issue(0, 0)

    def chunk(g, _):
        slot = lax.rem(g, 2)

        @pl.when(g + 1 < NCH)
        def _pref():
            issue(1 - slot, g + 1)
        drain(slot, g)

        @plsc.parallel_loop(0, NG, unroll=8)
        def group(i):
            sl = pl.ds(i * 16, 16)
            src16 = idx_sv[slot, sl]
            dst16 = idx_dv[slot, sl]
            ea16 = ea_v[slot, sl]
            for f in range(FPT):
                a16 = plsc.load_gather(a_rows, [rowf[f], src16])
                b16 = plsc.load_gather(b_rows, [rowf[f], dst16])
                r = jnp.maximum(a16 + b16 + ea16 * w1ef[f], 0.0)
                plsc.addupdate_scatter(acc, [rowf[f], dst16], r)
        return 0
    lax.fori_loop(0, NCH, chunk, 0)

    pltpu.sync_copy(acc, st_out.at[wid])


@functools.cache
def _sc_edge():
  return pl.kernel(
    _sc_edge_body,
    out_type=jax.ShapeDtypeStruct((NW, FPT, N), _F32),
    mesh=plsc.VectorSubcoreMesh(core_axis_name="c", subcore_axis_name="s",
                                num_cores=NC, num_subcores=NS),
    compiler_params=pltpu.CompilerParams(needs_layout_passes=False,
                                         internal_scratch_in_bytes=16 * 1024),
    scratch_types=[
        pltpu.VMEM((2, CH), jnp.int32),
        pltpu.VMEM((2, CH), jnp.int32),
        pltpu.VMEM((2, CH), _F32),
        pltpu.VMEM((FPT, N), _F32),
        pltpu.VMEM((FPT, N), _F32),
        pltpu.VMEM((FPT, N), _F32),
        pltpu.VMEM((H,), _F32),
        pltpu.SemaphoreType.DMA,
        pltpu.SemaphoreType.DMA,
    ],
  )


CHD = 2000
EPW = E // NW


def _sc_deg_body(dst_hbm, deg_out, idxd_v, dacc):
    c = lax.axis_index("c")
    s = lax.axis_index("s")
    wid = s * NC + c
    zero16 = jnp.zeros((16,), _F32)
    one16 = jnp.ones((16,), _F32)
    row0 = jnp.full((16,), 0, jnp.int32)

    def zrow(i, _):
        dacc[0, pl.ds(i * 16, 16)] = zero16
        return 0
    lax.fori_loop(0, N // 16, zrow, 0)

    def chunk(q, _):
        pltpu.sync_copy(dst_hbm.at[pl.ds(wid * EPW + q * CHD, CHD)], idxd_v)

        def group(i, _):
            dst16 = idxd_v[pl.ds(i * 16, 16)]
            plsc.addupdate_scatter(dacc, [row0, dst16], one16)
            return 0
        lax.fori_loop(0, CHD // 16, group, 0)
        return 0
    lax.fori_loop(0, EPW // CHD, chunk, 0)
    pltpu.sync_copy(dacc, deg_out.at[wid])


@functools.cache
def _sc_deg():
  return pl.kernel(
    _sc_deg_body,
    out_type=jax.ShapeDtypeStruct((NW, 1, N), _F32),
    mesh=plsc.VectorSubcoreMesh(core_axis_name="c", subcore_axis_name="s",
                                num_cores=NC, num_subcores=NS),
    compiler_params=pltpu.CompilerParams(needs_layout_passes=False,
                                         internal_scratch_in_bytes=32 * 1024),
    scratch_types=[
        pltpu.VMEM((CHD,), jnp.int32),
        pltpu.VMEM((1, N), _F32),
    ],
  )


def kernel(z, edge_index, edge_attr, batch, embed,
           msg_w1, msg_b1, msg_w2, msg_b2,
           upd_w1, upd_b1, upd_w2, upd_b2,
           head_w1, head_b1, head_w2, head_b2):
    z2 = z.astype(jnp.int32).reshape(N, 1)
    src = edge_index[0].astype(jnp.int32)
    dst = edge_index[1].astype(jnp.int32)
    ea = edge_attr.reshape(E).astype(jnp.float32)
    batch2 = batch.astype(jnp.int32).reshape(N, 1)
    emb_p = jnp.pad(embed, ((0, VPAD - embed.shape[0]), (0, 0)))

    w1a = [msg_w1[l, :H] for l in range(L)]
    w1b = [msg_w1[l, H:2 * H] for l in range(L)]
    w1e = [msg_w1[l, 2 * H] for l in range(L)]
    b1c = [msg_b1[l].reshape(H, 1) for l in range(L)]
    w2 = [msg_w2[l] for l in range(L)]
    b2 = [msg_b2[l].reshape(1, H) for l in range(L)]
    u1a = [upd_w1[l, :H] for l in range(L)]
    u1b = [upd_w1[l, H:] for l in range(L)]
    ub1 = [upd_b1[l].reshape(1, H) for l in range(L)]
    u2 = [upd_w2[l] for l in range(L)]
    ub2 = [upd_b2[l].reshape(1, H) for l in range(L)]

    h, at, bt = _tc_init(z2, emb_p, w1a[0], w1b[0], b1c[0])
    deg32 = _sc_deg()(dst).reshape(NW, N)
    out = None
    for l in range(L):
        st3 = _sc_edge()(at.reshape(NW, FPT, N), bt.reshape(NW, FPT, N),
                         src, dst, ea, w1e[l])
        st = st3.reshape(H, N)
        if l + 1 < L:
            h, at, bt = _tc_update(h, st, deg32, w2[l], b2[l], u1a[l], u1b[l],
                                   ub1[l], u2[l], ub2[l],
                                   w1a[l + 1], w1b[l + 1], b1c[l + 1])
        else:
            out = _tc_final(h, st, deg32, w2[l], b2[l], u1a[l], u1b[l],
                            ub1[l], u2[l], ub2[l], batch2,
                            head_w1, head_b1.reshape(1, H),
                            head_w2, head_b2.reshape(1, OUT))
    return out
